# baseline (speedup 1.0000x reference)
import functools

import jax
import jax.numpy as jnp
from jax import lax
from jax.experimental import pallas as pl
from jax.experimental.pallas import tpu as pltpu
from jax.experimental.pallas import tpu_sc as plsc

_GAMMA = 24.0
_B = 16384
_D = 128
_NW = 32
_BPW = _B // _NW
_C = 32
_NCHUNK = _BPW // _C
_L = 16
_G = _D // _L
_NREL = 500
_NTIME = 365


def _pair_hsum(a, b, lane):
    a = a + a.at[(lane + 8) & (_L - 1)].get(mode="promise_in_bounds")
    b = b + b.at[(lane + 8) & (_L - 1)].get(mode="promise_in_bounds")
    c = jnp.where(lane < 8, a, b)
    for sh in (4, 2, 1):
        perm = (lane & 8) | ((lane + sh) & 7)
        c = c + c.at[perm].get(mode="promise_in_bounds")
    return c


def _sc_body(head_hbm, rel_hbm, tail_hbm, time_hbm,
             ent_hbm, rel_emb_hbm, time_emb_hbm, out_hbm,
             hidx_v, ridx_v, tidx_v, tauidx_v, ridx_s, tauidx_s,
             h0, t0, h1, t1, rel_tbl, tau_tbl,
             out_v, sem0, sem1):
    wid = lax.axis_index("s") * 2 + lax.axis_index("c")
    base = wid * _BPW
    lane = lax.iota(jnp.int32, _L)

    pltpu.sync_copy(rel_emb_hbm, rel_tbl)
    pltpu.sync_copy(time_emb_hbm, tau_tbl)

    pltpu.sync_copy(head_hbm.at[pl.ds(base, _BPW)], hidx_v)
    pltpu.sync_copy(rel_hbm.at[pl.ds(base, _BPW)], ridx_v)
    pltpu.sync_copy(tail_hbm.at[pl.ds(base, _BPW)], tidx_v)
    pltpu.sync_copy(time_hbm.at[pl.ds(base, _BPW)], tauidx_v)

    def _to_smem(q, carry):
        rvec = ridx_v[pl.ds(q * _L, _L)]
        tvec = tauidx_v[pl.ds(q * _L, _L)]
        for e in range(_L):
            ridx_s[q * _L + e] = rvec[e]
            tauidx_s[q * _L + e] = tvec[e]
        return carry

    lax.fori_loop(0, _BPW // _L, _to_smem, 0)

    def _copies(j, bufs, sem):
        h_v, t_v = bufs
        sl = pl.ds(j * _C, _C)
        return (
            pltpu.make_async_copy(ent_hbm.at[hidx_v.at[sl]], h_v, sem),
            pltpu.make_async_copy(ent_hbm.at[tidx_v.at[sl]], t_v, sem),
        )

    def _issue(j, bufs, sem):
        for cp in _copies(j, bufs, sem):
            cp.start()

    def _drain(j, bufs, sem):
        for cp in _copies(j, bufs, sem):
            cp.wait()

    def _score(j, bufs):
        h_v, t_v = bufs

        def blk_body(b, carry):
            def row_pair(k, scores):
                i = b * _L + k
                tots = []
                for ii in (i, i + 8):
                    gi = j * _C + ii
                    ri = ridx_s[gi]
                    ti = tauidx_s[gi]
                    accs = [None] * 4
                    for g in range(_G):
                        sl = pl.ds(g * _L, _L)
                        term = jnp.abs((h_v[ii, sl] + rel_tbl[ri, sl])
                                       + (tau_tbl[ti, sl] - t_v[ii, sl]))
                        a = accs[g % 4]
                        accs[g % 4] = term if a is None else a + term
                    tots.append((accs[0] + accs[1]) + (accs[2] + accs[3]))
                c = _pair_hsum(tots[0], tots[1], lane)
                return jnp.where((lane & 7) == k, _GAMMA - c, scores)

            scores = lax.fori_loop(0, _L // 2, row_pair,
                                   jnp.zeros((_L,), jnp.float32))
            out_v[pl.ds(j * _C + b * _L, _L)] = scores
            return carry

        lax.fori_loop(0, _C // _L, blk_body, 0)

    set0 = (h0, t0)
    set1 = (h1, t1)

    _issue(0, set0, sem0)

    def m_body(m, carry):
        j0 = 2 * m
        _issue(j0 + 1, set1, sem1)
        _drain(j0, set0, sem0)
        _score(j0, set0)

        @pl.when(j0 + 2 < _NCHUNK)
        def _():
            _issue(j0 + 2, set0, sem0)

        _drain(j0 + 1, set1, sem1)
        _score(j0 + 1, set1)
        return carry

    lax.fori_loop(0, _NCHUNK // 2, m_body, 0)
    pltpu.sync_copy(out_v, out_hbm.at[pl.ds(base, _BPW)])


@functools.partial(
    pl.kernel,
    out_type=jax.ShapeDtypeStruct((_B,), jnp.float32),
    mesh=plsc.VectorSubcoreMesh(core_axis_name="c", subcore_axis_name="s"),
    scratch_types=[
        pltpu.VMEM((_BPW,), jnp.int32),
        pltpu.VMEM((_BPW,), jnp.int32),
        pltpu.VMEM((_BPW,), jnp.int32),
        pltpu.VMEM((_BPW,), jnp.int32),
        pltpu.SMEM((_BPW,), jnp.int32),
        pltpu.SMEM((_BPW,), jnp.int32),
        pltpu.VMEM((_C, _D), jnp.float32),
        pltpu.VMEM((_C, _D), jnp.float32),
        pltpu.VMEM((_C, _D), jnp.float32),
        pltpu.VMEM((_C, _D), jnp.float32),
        pltpu.VMEM((_NREL, _D), jnp.float32),
        pltpu.VMEM((_NTIME, _D), jnp.float32),
        pltpu.VMEM((_BPW,), jnp.float32),
        pltpu.SemaphoreType.DMA,
        pltpu.SemaphoreType.DMA,
    ],
)
def _sc_kernel(*refs):
    _sc_body(*refs)


def kernel(head_index, relation_index, tail_index, time_index,
           entity_embedding, relation_embedding, time_embedding):
    return _sc_kernel(head_index.astype(jnp.int32),
                      relation_index.astype(jnp.int32),
                      tail_index.astype(jnp.int32),
                      time_index.astype(jnp.int32),
                      entity_embedding, relation_embedding, time_embedding)


# speedup vs baseline: 1.1164x; 1.1164x over previous
import functools

import jax
import jax.numpy as jnp
from jax import lax
from jax.experimental import pallas as pl
from jax.experimental.pallas import tpu as pltpu
from jax.experimental.pallas import tpu_sc as plsc

_GAMMA = 24.0
_B = 16384
_D = 128
_NW = 32
_BPW = _B // _NW
_C = 64
_NCHUNK = _BPW // _C
_NBUF = 3
_L = 16
_G = _D // _L


def _pair_hsum(a, b, lane):
    a = a + a.at[(lane + 8) & (_L - 1)].get(mode="promise_in_bounds")
    b = b + b.at[(lane + 8) & (_L - 1)].get(mode="promise_in_bounds")
    c = jnp.where(lane < 8, a, b)
    for sh in (4, 2, 1):
        perm = (lane & 8) | ((lane + sh) & 7)
        c = c + c.at[perm].get(mode="promise_in_bounds")
    return c


def _sc_body(head_hbm, rel_hbm, tail_hbm, time_hbm,
             ent_hbm, rel_emb_hbm, time_emb_hbm, out_hbm,
             hidx_v, ridx_v, tidx_v, tauidx_v,
             *bufs_and_sems):
    bufs = [bufs_and_sems[4 * i:4 * i + 4] for i in range(_NBUF)]
    out_v = bufs_and_sems[4 * _NBUF]
    sems = bufs_and_sems[4 * _NBUF + 1:4 * _NBUF + 1 + _NBUF]

    wid = lax.axis_index("s") * 2 + lax.axis_index("c")
    base = wid * _BPW
    lane = lax.iota(jnp.int32, _L)

    pltpu.sync_copy(head_hbm.at[pl.ds(base, _BPW)], hidx_v)
    pltpu.sync_copy(rel_hbm.at[pl.ds(base, _BPW)], ridx_v)
    pltpu.sync_copy(tail_hbm.at[pl.ds(base, _BPW)], tidx_v)
    pltpu.sync_copy(time_hbm.at[pl.ds(base, _BPW)], tauidx_v)

    def _copies(j, buf, sem):
        h_v, r_v, t_v, tau_v = buf
        sl = pl.ds(j * _C, _C)
        return (
            pltpu.make_async_copy(ent_hbm.at[hidx_v.at[sl]], h_v, sem),
            pltpu.make_async_copy(ent_hbm.at[tidx_v.at[sl]], t_v, sem),
            pltpu.make_async_copy(rel_emb_hbm.at[ridx_v.at[sl]], r_v, sem),
            pltpu.make_async_copy(time_emb_hbm.at[tauidx_v.at[sl]], tau_v, sem),
        )

    def _score(j, buf):
        h_v, r_v, t_v, tau_v = buf

        def blk_body(b, carry):
            def one_row(i):
                accs = [None] * 4
                for g in range(_G):
                    sl = pl.ds(g * _L, _L)
                    term = jnp.abs((h_v[i, sl] + r_v[i, sl])
                                   + (tau_v[i, sl] - t_v[i, sl]))
                    a = accs[g % 4]
                    accs[g % 4] = term if a is None else a + term
                return (accs[0] + accs[1]) + (accs[2] + accs[3])

            def row_pair(k, scores):
                i = b * _L + k
                c = _pair_hsum(one_row(i), one_row(i + 8), lane)
                return jnp.where((lane & 7) == k, _GAMMA - c, scores)

            scores = lax.fori_loop(0, _L // 2, row_pair,
                                   jnp.zeros((_L,), jnp.float32))
            out_v[pl.ds(j * _C + b * _L, _L)] = scores
            return carry

        lax.fori_loop(0, _C // _L, blk_body, 0)

    for j in range(min(_NBUF, _NCHUNK)):
        for cp in _copies(j, bufs[j % _NBUF], sems[j % _NBUF]):
            cp.start()

    for j in range(_NCHUNK):
        for cp in _copies(j, bufs[j % _NBUF], sems[j % _NBUF]):
            cp.wait()
        _score(j, bufs[j % _NBUF])
        if j + _NBUF < _NCHUNK:
            for cp in _copies(j + _NBUF, bufs[j % _NBUF], sems[j % _NBUF]):
                cp.start()

    pltpu.sync_copy(out_v, out_hbm.at[pl.ds(base, _BPW)])


@functools.partial(
    pl.kernel,
    out_type=jax.ShapeDtypeStruct((_B,), jnp.float32),
    mesh=plsc.VectorSubcoreMesh(core_axis_name="c", subcore_axis_name="s"),
    scratch_types=(
        [pltpu.VMEM((_BPW,), jnp.int32)] * 4
        + [pltpu.VMEM((_C, _D), jnp.float32)] * (4 * _NBUF)
        + [pltpu.VMEM((_BPW,), jnp.float32)]
        + [pltpu.SemaphoreType.DMA] * _NBUF
    ),
)
def _sc_kernel(*refs):
    _sc_body(*refs)


def kernel(head_index, relation_index, tail_index, time_index,
           entity_embedding, relation_embedding, time_embedding):
    return _sc_kernel(head_index.astype(jnp.int32),
                      relation_index.astype(jnp.int32),
                      tail_index.astype(jnp.int32),
                      time_index.astype(jnp.int32),
                      entity_embedding, relation_embedding, time_embedding)


# speedup vs baseline: 1.2689x; 1.1366x over previous
import functools

import jax
import jax.numpy as jnp
from jax import lax
from jax.experimental import pallas as pl
from jax.experimental.pallas import tpu as pltpu
from jax.experimental.pallas import tpu_sc as plsc

_GAMMA = 24.0
_B = 16384
_D = 128
_NW = 32
_BPW = _B // _NW
_C = 64
_NCHUNK = _BPW // _C
_L = 16
_G = _D // _L


def _hsum_all_lanes(v, lane):
    for sh in (8, 4, 2, 1):
        perm = (lane + sh) & (_L - 1)
        v = v + v.at[perm].get(mode="promise_in_bounds")
    return v


def _sc_body(head_hbm, rel_hbm, tail_hbm, time_hbm,
             ent_hbm, rel_emb_hbm, time_emb_hbm, out_hbm,
             hidx_v, ridx_v, tidx_v, tauidx_v,
             h0, r0, t0, tau0, h1, r1, t1, tau1,
             out_v, sem0, sem1):
    wid = lax.axis_index("s") * 2 + lax.axis_index("c")
    base = wid * _BPW
    lane = lax.iota(jnp.int32, _L)

    pltpu.sync_copy(head_hbm.at[pl.ds(base, _BPW)], hidx_v)
    pltpu.sync_copy(rel_hbm.at[pl.ds(base, _BPW)], ridx_v)
    pltpu.sync_copy(tail_hbm.at[pl.ds(base, _BPW)], tidx_v)
    pltpu.sync_copy(time_hbm.at[pl.ds(base, _BPW)], tauidx_v)

    def _copies(j, bufs, sem):
        h_v, r_v, t_v, tau_v = bufs
        sl = pl.ds(j * _C, _C)
        return (
            pltpu.make_async_copy(ent_hbm.at[hidx_v.at[sl]], h_v, sem),
            pltpu.make_async_copy(ent_hbm.at[tidx_v.at[sl]], t_v, sem),
            pltpu.make_async_copy(rel_emb_hbm.at[ridx_v.at[sl]], r_v, sem),
            pltpu.make_async_copy(time_emb_hbm.at[tauidx_v.at[sl]], tau_v, sem),
        )

    def _issue(j, bufs, sem):
        for cp in _copies(j, bufs, sem):
            cp.start()

    def _drain(j, bufs, sem):
        for cp in _copies(j, bufs, sem):
            cp.wait()

    def _score(j, bufs):
        h_v, r_v, t_v, tau_v = bufs

        def blk_body(b, carry):
            def row_body(k, scores):
                i = b * _L + k
                accs = [None] * 4
                for g in range(_G):
                    sl = pl.ds(g * _L, _L)
                    term = jnp.abs((h_v[i, sl] + r_v[i, sl])
                                   + (tau_v[i, sl] - t_v[i, sl]))
                    a = accs[g % 4]
                    accs[g % 4] = term if a is None else a + term
                acc = (accs[0] + accs[1]) + (accs[2] + accs[3])
                tot = _hsum_all_lanes(acc, lane)
                return jnp.where(lane == k, _GAMMA - tot, scores)

            scores = lax.fori_loop(0, _L, row_body,
                                   jnp.zeros((_L,), jnp.float32))
            out_v[pl.ds(j * _C + b * _L, _L)] = scores
            return carry

        lax.fori_loop(0, _C // _L, blk_body, 0)

    set0 = (h0, r0, t0, tau0)
    set1 = (h1, r1, t1, tau1)

    _issue(0, set0, sem0)

    def m_body(m, carry):
        j0 = 2 * m
        _issue(j0 + 1, set1, sem1)
        _drain(j0, set0, sem0)
        _score(j0, set0)

        @pl.when(j0 + 2 < _NCHUNK)
        def _():
            _issue(j0 + 2, set0, sem0)

        _drain(j0 + 1, set1, sem1)
        _score(j0 + 1, set1)
        return carry

    lax.fori_loop(0, _NCHUNK // 2, m_body, 0)
    pltpu.sync_copy(out_v, out_hbm.at[pl.ds(base, _BPW)])


@functools.partial(
    pl.kernel,
    out_type=jax.ShapeDtypeStruct((_B,), jnp.float32),
    mesh=plsc.VectorSubcoreMesh(core_axis_name="c", subcore_axis_name="s"),
    scratch_types=(
        [pltpu.VMEM((_BPW,), jnp.int32)] * 4
        + [pltpu.VMEM((_C, _D), jnp.float32)] * 8
        + [pltpu.VMEM((_BPW,), jnp.float32)]
        + [pltpu.SemaphoreType.DMA] * 2
    ),
)
def _sc_kernel(*refs):
    _sc_body(*refs)


def kernel(head_index, relation_index, tail_index, time_index,
           entity_embedding, relation_embedding, time_embedding):
    return _sc_kernel(head_index.astype(jnp.int32),
                      relation_index.astype(jnp.int32),
                      tail_index.astype(jnp.int32),
                      time_index.astype(jnp.int32),
                      entity_embedding, relation_embedding, time_embedding)
